# baseline (speedup 1.0000x reference)
import jax
import jax.numpy as jnp
from jax.experimental import pallas as pl
from jax.experimental.pallas import tpu as pltpu

_CACHE_SIZE = 4096.0
_B = 64
_BB = 8
_HALF = 8
_ROPE = 10000.0


def _rotate_kernel(sinks_ref, pos_ref, kc_ref, out_ref):
    base = pl.program_id(0) * _BB
    sinks = sinks_ref[...]
    pos = pos_ref[...].astype(jnp.float32)
    evict = jnp.maximum(pos - _CACHE_SIZE, 0.0).astype(jnp.int32).astype(jnp.float32)

    blk = base + jax.lax.broadcasted_iota(jnp.int32, (_BB, _B), 0)
    match = blk == sinks
    p_masked = jnp.where(match, evict, 0.0)
    needs_rot = jnp.sum(p_masked) != 0.0

    @pl.when(needs_rot)
    def _rotate():
        p_total = jnp.sum(p_masked, axis=1, keepdims=True)
        j = jax.lax.broadcasted_iota(jnp.int32, (_HALF, 128), 0)
        lane = jax.lax.broadcasted_iota(jnp.int32, (_HALF, 128), 1)
        f = (j * 8 + lane % 8).astype(jnp.float32)
        inv_freq = 1.0 / (_ROPE ** (f * 2.0 / 128.0))

        angle = p_total[:, :, None] * inv_freq[None, :, :]
        cos = jnp.cos(angle)[:, None, :, :]
        sin = jnp.sin(angle)[:, None, :, :]

        data = kc_ref[...]
        a = data[:, :, :_HALF, :]
        b = data[:, :, _HALF:, :]
        out_ref[:, :, :_HALF, :] = a * cos - b * sin
        out_ref[:, :, _HALF:, :] = b * cos + a * sin

    @pl.when(jnp.logical_not(needs_rot))
    def _copy():
        out_ref[...] = kc_ref[...]


def kernel(key_cache, block_tables, context_lens, positions):
    del context_lens
    nb, h, dx, bs, x = key_cache.shape
    kc = key_cache.reshape(nb, h, dx, bs * x)
    sinks = block_tables[:, 0].reshape(1, _B)
    pos = positions.reshape(1, _B)
    out = pl.pallas_call(
        _rotate_kernel,
        grid=(nb // _BB,),
        in_specs=[
            pl.BlockSpec((1, _B), lambda i: (0, 0)),
            pl.BlockSpec((1, _B), lambda i: (0, 0)),
            pl.BlockSpec((_BB, h, dx, bs * x), lambda i: (i, 0, 0, 0)),
        ],
        out_specs=pl.BlockSpec((_BB, h, dx, bs * x), lambda i: (i, 0, 0, 0)),
        out_shape=jax.ShapeDtypeStruct((nb, h, dx, bs * x), key_cache.dtype),
        compiler_params=pltpu.CompilerParams(
            dimension_semantics=("parallel",),
        ),
    )(sinks, pos, kc)
    return out.reshape(nb, h, dx, bs, x)


# speedup vs baseline: 1.2436x; 1.2436x over previous
import jax
import jax.numpy as jnp
from jax.experimental import pallas as pl
from jax.experimental.pallas import tpu as pltpu

_CACHE_SIZE = 4096.0
_B = 64
_BB = 32
_HALF = 8
_ROPE = 10000.0


def _rotate_kernel(sinks_ref, pos_ref, kc_ref, out_ref):
    base = pl.program_id(0) * _BB
    sinks = sinks_ref[...]
    pos = pos_ref[...].astype(jnp.float32)
    evict = jnp.maximum(pos - _CACHE_SIZE, 0.0).astype(jnp.int32).astype(jnp.float32)

    blk = base + jax.lax.broadcasted_iota(jnp.int32, (_BB, _B), 0)
    match = blk == sinks
    p_masked = jnp.where(match, evict, 0.0)
    needs_rot = jnp.sum(p_masked) != 0.0

    @pl.when(needs_rot)
    def _rotate():
        p_total = jnp.sum(p_masked, axis=1, keepdims=True)
        j = jax.lax.broadcasted_iota(jnp.int32, (_HALF, 128), 0)
        lane = jax.lax.broadcasted_iota(jnp.int32, (_HALF, 128), 1)
        f = (j * 8 + lane % 8).astype(jnp.float32)
        inv_freq = 1.0 / (_ROPE ** (f * 2.0 / 128.0))

        angle = p_total[:, :, None] * inv_freq[None, :, :]
        cos = jnp.cos(angle)[:, None, :, :]
        sin = jnp.sin(angle)[:, None, :, :]

        data = kc_ref[...]
        a = data[:, :, :_HALF, :]
        b = data[:, :, _HALF:, :]
        out_ref[:, :, :_HALF, :] = a * cos - b * sin
        out_ref[:, :, _HALF:, :] = b * cos + a * sin

    @pl.when(jnp.logical_not(needs_rot))
    def _copy():
        out_ref[...] = kc_ref[...]


def kernel(key_cache, block_tables, context_lens, positions):
    del context_lens
    nb, h, dx, bs, x = key_cache.shape
    kc = key_cache.reshape(nb, h, dx, bs * x)
    sinks = block_tables[:, 0].reshape(1, _B)
    pos = positions.reshape(1, _B)
    out = pl.pallas_call(
        _rotate_kernel,
        grid=(nb // _BB,),
        in_specs=[
            pl.BlockSpec((1, _B), lambda i: (0, 0)),
            pl.BlockSpec((1, _B), lambda i: (0, 0)),
            pl.BlockSpec((_BB, h, dx, bs * x), lambda i: (i, 0, 0, 0)),
        ],
        out_specs=pl.BlockSpec((_BB, h, dx, bs * x), lambda i: (i, 0, 0, 0)),
        out_shape=jax.ShapeDtypeStruct((nb, h, dx, bs * x), key_cache.dtype),
        compiler_params=pltpu.CompilerParams(
            dimension_semantics=("parallel",),
        ),
    )(sinks, pos, kc)
    return out.reshape(nb, h, dx, bs, x)


# speedup vs baseline: 1.2852x; 1.0334x over previous
import jax
import jax.numpy as jnp
from jax.experimental import pallas as pl
from jax.experimental.pallas import tpu as pltpu

_CACHE_SIZE = 4096.0
_B = 64
_BB = 64
_HALF = 8
_ROPE = 10000.0


def _rotate_kernel(sinks_ref, pos_ref, kc_ref, out_ref):
    base = pl.program_id(0) * _BB
    sinks = sinks_ref[...]
    pos = pos_ref[...].astype(jnp.float32)
    evict = jnp.maximum(pos - _CACHE_SIZE, 0.0).astype(jnp.int32).astype(jnp.float32)

    blk = base + jax.lax.broadcasted_iota(jnp.int32, (_BB, _B), 0)
    match = blk == sinks
    p_masked = jnp.where(match, evict, 0.0)
    needs_rot = jnp.sum(p_masked) != 0.0

    @pl.when(needs_rot)
    def _rotate():
        p_total = jnp.sum(p_masked, axis=1, keepdims=True)
        j = jax.lax.broadcasted_iota(jnp.int32, (_HALF, 128), 0)
        lane = jax.lax.broadcasted_iota(jnp.int32, (_HALF, 128), 1)
        f = (j * 8 + lane % 8).astype(jnp.float32)
        inv_freq = 1.0 / (_ROPE ** (f * 2.0 / 128.0))

        angle = p_total[:, :, None] * inv_freq[None, :, :]
        cos = jnp.cos(angle)[:, None, :, :]
        sin = jnp.sin(angle)[:, None, :, :]

        data = kc_ref[...]
        a = data[:, :, :_HALF, :]
        b = data[:, :, _HALF:, :]
        out_ref[:, :, :_HALF, :] = a * cos - b * sin
        out_ref[:, :, _HALF:, :] = b * cos + a * sin

    @pl.when(jnp.logical_not(needs_rot))
    def _copy():
        out_ref[...] = kc_ref[...]


def kernel(key_cache, block_tables, context_lens, positions):
    del context_lens
    nb, h, dx, bs, x = key_cache.shape
    kc = key_cache.reshape(nb, h, dx, bs * x)
    sinks = block_tables[:, 0].reshape(1, _B)
    pos = positions.reshape(1, _B)
    out = pl.pallas_call(
        _rotate_kernel,
        grid=(nb // _BB,),
        in_specs=[
            pl.BlockSpec((1, _B), lambda i: (0, 0)),
            pl.BlockSpec((1, _B), lambda i: (0, 0)),
            pl.BlockSpec((_BB, h, dx, bs * x), lambda i: (i, 0, 0, 0)),
        ],
        out_specs=pl.BlockSpec((_BB, h, dx, bs * x), lambda i: (i, 0, 0, 0)),
        out_shape=jax.ShapeDtypeStruct((nb, h, dx, bs * x), key_cache.dtype),
        compiler_params=pltpu.CompilerParams(
            dimension_semantics=("parallel",),
        ),
    )(sinks, pos, kc)
    return out.reshape(nb, h, dx, bs, x)


# speedup vs baseline: 1.3132x; 1.0218x over previous
import jax
import jax.numpy as jnp
from jax.experimental import pallas as pl
from jax.experimental.pallas import tpu as pltpu

_CACHE_SIZE = 4096.0
_B = 64
_BB = 128
_HALF = 8
_ROPE = 10000.0


def _rotate_kernel(sinks_ref, pos_ref, kc_ref, out_ref):
    base = pl.program_id(0) * _BB
    sinks = sinks_ref[...]
    pos = pos_ref[...].astype(jnp.float32)
    evict = jnp.maximum(pos - _CACHE_SIZE, 0.0).astype(jnp.int32).astype(jnp.float32)

    blk = base + jax.lax.broadcasted_iota(jnp.int32, (_BB, _B), 0)
    match = blk == sinks
    p_masked = jnp.where(match, evict, 0.0)
    needs_rot = jnp.sum(p_masked) != 0.0

    @pl.when(needs_rot)
    def _rotate():
        p_total = jnp.sum(p_masked, axis=1, keepdims=True)
        j = jax.lax.broadcasted_iota(jnp.int32, (_HALF, 128), 0)
        lane = jax.lax.broadcasted_iota(jnp.int32, (_HALF, 128), 1)
        f = (j * 8 + lane % 8).astype(jnp.float32)
        inv_freq = 1.0 / (_ROPE ** (f * 2.0 / 128.0))

        angle = p_total[:, :, None] * inv_freq[None, :, :]
        cos = jnp.cos(angle)[:, None, :, :]
        sin = jnp.sin(angle)[:, None, :, :]

        data = kc_ref[...]
        a = data[:, :, :_HALF, :]
        b = data[:, :, _HALF:, :]
        out_ref[:, :, :_HALF, :] = a * cos - b * sin
        out_ref[:, :, _HALF:, :] = b * cos + a * sin

    @pl.when(jnp.logical_not(needs_rot))
    def _copy():
        out_ref[...] = kc_ref[...]


def kernel(key_cache, block_tables, context_lens, positions):
    del context_lens
    nb, h, dx, bs, x = key_cache.shape
    kc = key_cache.reshape(nb, h, dx, bs * x)
    sinks = block_tables[:, 0].reshape(1, _B)
    pos = positions.reshape(1, _B)
    out = pl.pallas_call(
        _rotate_kernel,
        grid=(nb // _BB,),
        in_specs=[
            pl.BlockSpec((1, _B), lambda i: (0, 0)),
            pl.BlockSpec((1, _B), lambda i: (0, 0)),
            pl.BlockSpec((_BB, h, dx, bs * x), lambda i: (i, 0, 0, 0)),
        ],
        out_specs=pl.BlockSpec((_BB, h, dx, bs * x), lambda i: (i, 0, 0, 0)),
        out_shape=jax.ShapeDtypeStruct((nb, h, dx, bs * x), key_cache.dtype),
        compiler_params=pltpu.CompilerParams(
            dimension_semantics=("parallel",),
        ),
    )(sinks, pos, kc)
    return out.reshape(nb, h, dx, bs, x)
